# baseline (speedup 1.0000x reference)
import jax
import jax.numpy as jnp
from jax import lax
from jax.experimental import pallas as pl
from jax.experimental.pallas import tpu as pltpu

_B, _R = 4, 1000
_RP = 1024
_NKEEP = 64
_H, _W, _C = 200, 200, 256
_TILE_R = 64
_TILE_C = 72
_POOL = 3


def _nms_body(roi_s_ref, roi_t_ref, out_ref, m_ref):
    f32 = jnp.float32
    xj = roi_t_ref[:, 0:1, :]
    yj = roi_t_ref[:, 1:2, :]
    wj = roi_t_ref[:, 2:3, :]
    hj = roi_t_ref[:, 3:4, :]
    jlane = lax.broadcasted_iota(jnp.int32, (_B, 8, _RP), 2)

    def mk_chunk(ci, carry):
        c0 = ci * 8
        blk = roi_s_ref[:, pl.ds(c0, 8), :]
        xi = blk[:, :, 0:1]
        yi = blk[:, :, 1:2]
        wi = blk[:, :, 2:3]
        hi = blk[:, :, 3:4]
        x1 = jnp.maximum(xi, xj)
        y1 = jnp.maximum(yi, yj)
        x2 = jnp.minimum(xi + wi, xj + wj)
        y2 = jnp.minimum(yi + hi, yj + hj)
        inter = jnp.maximum(0.0, x2 - x1) * jnp.maximum(0.0, y2 - y1)
        union = wi * hi + wj * hj - inter
        iou = inter / union
        isub = c0 + lax.broadcasted_iota(jnp.int32, (_B, 8, _RP), 1)
        m = jnp.where((iou > 0.4) & (jlane > isub) & (jlane < _R),
                      f32(1.0), f32(0.0))
        m_ref[:, pl.ds(c0, 8), :] = m
        return carry

    lax.fori_loop(0, _R // 8, mk_chunk, 0)

    sup = jnp.zeros((_B, _RP), f32)
    for p in range(_R - 1):
        row = m_ref[:, p, :]
        piv = sup[:, p:p + 1]
        sup = jnp.maximum(sup, row * (1.0 - piv))

    jl2 = lax.broadcasted_iota(jnp.int32, (_B, _RP), 1)
    valid = jnp.where(jl2 < _R, 1.0 - sup, f32(0.0))
    c = valid
    sh = 1
    while sh < _RP:
        z = jnp.zeros((_B, sh), f32)
        c = c + jnp.concatenate([z, c[:, : _RP - sh]], axis=1)
        sh *= 2
    rank0 = c - 1.0

    kio = lax.broadcasted_iota(jnp.int32, (_B, _NKEEP, _RP), 1)
    rank0i = rank0.astype(jnp.int32)
    oh = jnp.where(
        (valid[:, None, :] > 0.5) & (rank0i[:, None, :] == kio),
        f32(1.0), f32(0.0))
    found = jnp.sum(oh, axis=2)

    comps = []
    for d in range(4):
        cj = roi_t_ref[:, d:d + 1, :]
        s = jnp.sum(oh * cj, axis=2)
        s = jnp.where(found > 0.5, s, cj[:, :, _R - 1])
        comps.append(s)
    bx, by, bw, bh = comps

    xmin = jnp.maximum(0.0, bx).astype(jnp.int32)
    ymin = jnp.maximum(0.0, by).astype(jnp.int32)
    xmax = jnp.minimum(float(_W), bx + bw).astype(jnp.int32)
    ymax = jnp.minimum(float(_H), by + bh).astype(jnp.int32)

    def clip_axis(mn, mx, size, ps):
        pad = ps - (mx - mn)
        fix_min = mn < pad // 2
        fix_max = size - mx < (1 + pad) // 2
        pos = pad > 0
        sym = pos & (~(fix_min | fix_max))
        omin = jnp.where(sym, mn - pad // 2, mn)
        omax = jnp.where(sym, mx + (1 + pad) // 2, mx)
        omin = jnp.where(pos & fix_min, 0, omin)
        omax = jnp.where(pos & fix_min, ps, omax)
        omin = jnp.where(pos & fix_max, size - ps, omin)
        omax = jnp.where(pos & fix_max, size, omax)
        return omin, omax

    xo0, xo1 = clip_axis(xmin, xmax, _W, _POOL)
    yo0, yo1 = clip_axis(ymin, ymax, _H, _POOL)
    out_ref[...] = jnp.stack([xo0, yo0, xo1 - xo0, yo1 - yo0], axis=-1)


def _pool_body(roi_smem, feat_hbm, out_ref, buf, sem):
    n = pl.program_id(0)
    neg = jnp.float32(-jnp.inf)

    def coords(m):
        base = m * 4
        return (m // _NKEEP, roi_smem[base], roi_smem[base + 1],
                roi_smem[base + 2], roi_smem[base + 3])

    def dma_chunks(m, slot):
        b, x, y, w, h = coords(m)
        yc = jnp.minimum(y, _H - _TILE_R)
        xc = jnp.minimum((x // 8) * 8, _W - _TILE_C)
        dy = y - yc
        dx = x - xc
        out = []
        for k in range(_TILE_R // 16):
            for c in range(_TILE_C // 24):
                cond = ((16 * k < dy + h) & (16 * k + 16 > dy)
                        & (24 * c < dx + w) & (24 * c + 24 > dx))
                desc = pltpu.make_async_copy(
                    feat_hbm.at[b, pl.ds(yc + 16 * k, 16),
                                pl.ds(xc + 24 * c, 24), :],
                    buf.at[slot, pl.ds(16 * k, 16), pl.ds(24 * c, 24), :],
                    sem.at[slot])
                out.append((cond, desc))
        return out

    @pl.when(n == 0)
    def _():
        for cond, desc in dma_chunks(0, 0):
            @pl.when(cond)
            def _(desc=desc):
                desc.start()

    @pl.when(n + 1 < _B * _NKEEP)
    def _():
        for cond, desc in dma_chunks(n + 1, (n + 1) % 2):
            @pl.when(cond)
            def _(desc=desc):
                desc.start()

    for cond, desc in dma_chunks(n, n % 2):
        @pl.when(cond)
        def _(desc=desc):
            desc.wait()

    slot = n % 2
    b, x, y, w, h = coords(n)
    yc = jnp.minimum(y, _H - _TILE_R)
    xc = jnp.minimum((x // 8) * 8, _W - _TILE_C)
    dy = y - yc
    dx = x - xc
    hs = h // _POOL
    ws = w // _POOL
    rlo = (dy, dy + hs, dy + 2 * hs)
    rhi = (dy + hs, dy + 2 * hs, dy + h)
    clo = (dx, dx + ws, dx + 2 * ws)
    chi = (dx + ws, dx + 2 * ws, dx + w)

    acc0 = jnp.full((_TILE_C, _C), neg, jnp.float32)

    def row_max(r, a):
        return jnp.maximum(a, buf[slot, r])

    cio = lax.broadcasted_iota(jnp.int32, (_TILE_C, 1), 0)
    for i in range(3):
        rows_i = lax.fori_loop(rlo[i], rhi[i], row_max, acc0)
        for j in range(3):
            cm = (cio >= clo[j]) & (cio < chi[j])
            out_ref[0, 0, i, j] = jnp.max(jnp.where(cm, rows_i, neg), axis=0)


def kernel(features, roi):
    roi_t = jnp.transpose(roi, (0, 2, 1))
    roi_t = jnp.pad(roi_t, ((0, 0), (0, 0), (0, _RP - _R)))

    roi_clipped = pl.pallas_call(
        _nms_body,
        out_shape=jax.ShapeDtypeStruct((_B, _NKEEP, 4), jnp.int32),
        in_specs=[
            pl.BlockSpec(memory_space=pltpu.VMEM),
            pl.BlockSpec(memory_space=pltpu.VMEM),
        ],
        out_specs=pl.BlockSpec(memory_space=pltpu.VMEM),
        scratch_shapes=[pltpu.VMEM((_B, _R, _RP), jnp.float32)],
    )(roi, roi_t)

    roi_flat = roi_clipped.reshape(-1)

    pooled = pl.pallas_call(
        _pool_body,
        grid=(_B * _NKEEP,),
        out_shape=jax.ShapeDtypeStruct((_B, _NKEEP, _POOL, _POOL, _C),
                                       jnp.float32),
        in_specs=[
            pl.BlockSpec(memory_space=pltpu.SMEM),
            pl.BlockSpec(memory_space=pl.ANY),
        ],
        out_specs=pl.BlockSpec(
            (1, 1, _POOL, _POOL, _C),
            lambda n: (n // _NKEEP, n % _NKEEP, 0, 0, 0)),
        scratch_shapes=[
            pltpu.VMEM((2, _TILE_R, _TILE_C, _C), jnp.float32),
            pltpu.SemaphoreType.DMA((2,)),
        ],
        compiler_params=pltpu.CompilerParams(
            dimension_semantics=("arbitrary",)),
    )(roi_flat, features)

    return pooled, roi_clipped


# speedup vs baseline: 1.2374x; 1.2374x over previous
import jax
import jax.numpy as jnp
from jax import lax
from jax.experimental import pallas as pl
from jax.experimental.pallas import tpu as pltpu
from jax.experimental.pallas import tpu_sc as plsc

_B, _R = 4, 1000
_RP = 1024
_NKEEP = 64
_H, _W, _C = 200, 200, 256
_TILE_R = 64
_TILE_C = 72
_POOL = 3


def _nms_build_body(roi_s_ref, roi_t_ref, mp_ref, m_ref):
    f32 = jnp.float32
    xj = roi_t_ref[:, 0:1, :]
    yj = roi_t_ref[:, 1:2, :]
    wj = roi_t_ref[:, 2:3, :]
    hj = roi_t_ref[:, 3:4, :]
    jlane = lax.broadcasted_iota(jnp.int32, (_B, 8, _RP), 2)

    def mk_chunk(ci, carry):
        c0 = ci * 8
        blk = roi_s_ref[:, pl.ds(c0, 8), :]
        xi = blk[:, :, 0:1]
        yi = blk[:, :, 1:2]
        wi = blk[:, :, 2:3]
        hi = blk[:, :, 3:4]
        x1 = jnp.maximum(xi, xj)
        y1 = jnp.maximum(yi, yj)
        x2 = jnp.minimum(xi + wi, xj + wj)
        y2 = jnp.minimum(yi + hi, yj + hj)
        inter = jnp.maximum(0.0, x2 - x1) * jnp.maximum(0.0, y2 - y1)
        union = wi * hi + wj * hj - inter
        iou = inter / union
        isub = c0 + lax.broadcasted_iota(jnp.int32, (_B, 8, _RP), 1)
        m = jnp.where((iou > 0.4) & (jlane > isub) & (jlane < _R),
                      f32(1.0), f32(0.0))
        m_ref[:, pl.ds(c0, 8), :] = m
        return carry

    lax.fori_loop(0, _R // 8, mk_chunk, 0)

    pj = lax.broadcasted_iota(jnp.int32, (_RP, 32), 0)
    pt = lax.broadcasted_iota(jnp.int32, (_RP, 32), 1)
    word = pj // 32
    bit = pj % 32
    sel_lo = (word == pt) & (bit < 16)
    sel_hi = (word == pt) & (bit >= 16)
    pmat_lo = jnp.where(sel_lo, 1 << jnp.where(sel_lo, bit, 0),
                        0).astype(f32)
    pmat_hi = jnp.where(sel_hi, 1 << jnp.where(sel_hi, bit - 16, 0),
                        0).astype(f32)

    def pack_slab(s, carry):
        for b in range(_B):
            slab = m_ref[b, pl.ds(s * 128, 128), :]
            lo = lax.dot_general(
                slab, pmat_lo, (((1,), (0,)), ((), ())),
                preferred_element_type=f32).astype(jnp.int32)
            hi = lax.dot_general(
                slab, pmat_hi, (((1,), (0,)), ((), ())),
                preferred_element_type=f32).astype(jnp.int32)
            mp_ref[pl.ds(b, 1), pl.ds(s * 128, 128), :] = (
                (lo | (hi << 16)).reshape(1, 128, 32))
        return carry

    lax.fori_loop(0, _RP // 128, pack_slab, 0)


def _dyn_gather16(v, idx):
    dnums = lax.GatherDimensionNumbers(
        offset_dims=(), collapsed_slice_dims=(0,), start_index_map=(0,))
    return lax.gather(v, idx[:, None], dnums, (1,),
                      mode=lax.GatherScatterMode.PROMISE_IN_BOUNDS)


def _sc_scan_body(mp_hbm, sup_hbm, mrows_v, supout_v):
    cid = lax.axis_index("c")
    sid = lax.axis_index("s")

    @pl.when((cid == 0) & (sid < _B))
    def _():
        unroll = 128

        def chunk(i, carry):
            s0, s1 = carry
            pltpu.sync_copy(mp_hbm.at[sid, pl.ds(i * unroll, unroll)],
                            mrows_v)
            for t in range(unroll):
                p = i * unroll + t
                row0 = mrows_v[t, pl.ds(0, 16)]
                row1 = mrows_v[t, pl.ds(16, 16)]
                wp = i * (unroll // 32) + t // 32
                wpv = jnp.full((16,), wp, jnp.int32)
                w0v = _dyn_gather16(s0, jnp.minimum(wpv, 15))
                w1v = _dyn_gather16(s1, jnp.maximum(wpv - 16, 0))
                a = ((wpv - 16) >> 31) & 1
                wv = w0v * a + w1v * (1 - a)
                bp = t % 32
                kb = 1 - ((wv >> bp) & 1)
                pb = jnp.full((16,), p, jnp.int32)
                inb = ((pb - (_R - 1)) >> 31) & 1
                m = kb * inb
                s0 = s0 | (row0 * m)
                s1 = s1 | (row1 * m)
            return s0, s1

        z = jnp.zeros((16,), jnp.int32)
        s0, s1 = lax.fori_loop(0, _RP // unroll, chunk, (z, z))
        supout_v[pl.ds(0, 16)] = s0
        supout_v[pl.ds(16, 16)] = s1
        pltpu.sync_copy(supout_v, sup_hbm.at[sid])


def _select_body(sup_ref, roi_t_ref, out_ref):
    f32 = jnp.float32
    sup = sup_ref[...]

    parts = [jnp.broadcast_to(sup[:, t:t + 1], (_B, 32)) for t in range(32)]
    rep = jnp.concatenate(parts, axis=1)
    jl2 = lax.broadcasted_iota(jnp.int32, (_B, _RP), 1)
    supbit = (rep >> (jl2 % 32)) & 1

    valid = jnp.where((jl2 < _R) & (supbit == 0), f32(1.0), f32(0.0))
    c = valid
    sh = 1
    while sh < _RP:
        z = jnp.zeros((_B, sh), f32)
        c = c + jnp.concatenate([z, c[:, : _RP - sh]], axis=1)
        sh *= 2
    rank0 = c - 1.0

    kio = lax.broadcasted_iota(jnp.int32, (_B, _NKEEP, _RP), 1)
    rank0i = rank0.astype(jnp.int32)
    oh = jnp.where(
        (valid[:, None, :] > 0.5) & (rank0i[:, None, :] == kio),
        f32(1.0), f32(0.0))
    found = jnp.sum(oh, axis=2)

    comps = []
    for d in range(4):
        cj = roi_t_ref[:, d:d + 1, :]
        s = jnp.sum(oh * cj, axis=2)
        s = jnp.where(found > 0.5, s, cj[:, :, _R - 1])
        comps.append(s)
    bx, by, bw, bh = comps

    xmin = jnp.maximum(0.0, bx).astype(jnp.int32)
    ymin = jnp.maximum(0.0, by).astype(jnp.int32)
    xmax = jnp.minimum(float(_W), bx + bw).astype(jnp.int32)
    ymax = jnp.minimum(float(_H), by + bh).astype(jnp.int32)

    def clip_axis(mn, mx, size, ps):
        pad = ps - (mx - mn)
        fix_min = mn < pad // 2
        fix_max = size - mx < (1 + pad) // 2
        pos = pad > 0
        sym = pos & (~(fix_min | fix_max))
        omin = jnp.where(sym, mn - pad // 2, mn)
        omax = jnp.where(sym, mx + (1 + pad) // 2, mx)
        omin = jnp.where(pos & fix_min, 0, omin)
        omax = jnp.where(pos & fix_min, ps, omax)
        omin = jnp.where(pos & fix_max, size - ps, omin)
        omax = jnp.where(pos & fix_max, size, omax)
        return omin, omax

    xo0, xo1 = clip_axis(xmin, xmax, _W, _POOL)
    yo0, yo1 = clip_axis(ymin, ymax, _H, _POOL)
    out_ref[...] = jnp.stack([xo0, yo0, xo1 - xo0, yo1 - yo0], axis=-1)


def _pool_body(roi_smem, feat_hbm, out_ref, buf, sem):
    n = pl.program_id(0)
    neg = jnp.float32(-jnp.inf)

    def coords(m):
        base = m * 4
        return (m // _NKEEP, roi_smem[base], roi_smem[base + 1],
                roi_smem[base + 2], roi_smem[base + 3])

    def dma_chunks(m, slot):
        b, x, y, w, h = coords(m)
        yc = jnp.minimum(y, _H - _TILE_R)
        xc = jnp.minimum((x // 8) * 8, _W - _TILE_C)
        dy = y - yc
        dx = x - xc
        out = []
        for k in range(_TILE_R // 16):
            for c in range(_TILE_C // 24):
                cond = ((16 * k < dy + h) & (16 * k + 16 > dy)
                        & (24 * c < dx + w) & (24 * c + 24 > dx))
                desc = pltpu.make_async_copy(
                    feat_hbm.at[b, pl.ds(yc + 16 * k, 16),
                                pl.ds(xc + 24 * c, 24), :],
                    buf.at[slot, pl.ds(16 * k, 16), pl.ds(24 * c, 24), :],
                    sem.at[slot])
                out.append((cond, desc))
        return out

    @pl.when(n == 0)
    def _():
        for cond, desc in dma_chunks(0, 0):
            @pl.when(cond)
            def _(desc=desc):
                desc.start()

    @pl.when(n + 1 < _B * _NKEEP)
    def _():
        for cond, desc in dma_chunks(n + 1, (n + 1) % 2):
            @pl.when(cond)
            def _(desc=desc):
                desc.start()

    for cond, desc in dma_chunks(n, n % 2):
        @pl.when(cond)
        def _(desc=desc):
            desc.wait()

    slot = n % 2
    b, x, y, w, h = coords(n)
    yc = jnp.minimum(y, _H - _TILE_R)
    xc = jnp.minimum((x // 8) * 8, _W - _TILE_C)
    dy = y - yc
    dx = x - xc
    hs = h // _POOL
    ws = w // _POOL
    rlo = (dy, dy + hs, dy + 2 * hs)
    rhi = (dy + hs, dy + 2 * hs, dy + h)
    clo = (dx, dx + ws, dx + 2 * ws)
    chi = (dx + ws, dx + 2 * ws, dx + w)

    acc0 = jnp.full((_TILE_C, _C), neg, jnp.float32)

    def row_max(r, a):
        return jnp.maximum(a, buf[slot, r])

    cio = lax.broadcasted_iota(jnp.int32, (_TILE_C, 1), 0)
    for i in range(3):
        rows_i = lax.fori_loop(rlo[i], rhi[i], row_max, acc0)
        for j in range(3):
            cm = (cio >= clo[j]) & (cio < chi[j])
            out_ref[0, 0, i, j] = jnp.max(jnp.where(cm, rows_i, neg), axis=0)


def kernel(features, roi):
    roi_t = jnp.transpose(roi, (0, 2, 1))
    roi_t = jnp.pad(roi_t, ((0, 0), (0, 0), (0, _RP - _R)))

    mp = pl.pallas_call(
        _nms_build_body,
        out_shape=jax.ShapeDtypeStruct((_B, _RP, 32), jnp.int32),
        in_specs=[
            pl.BlockSpec(memory_space=pltpu.VMEM),
            pl.BlockSpec(memory_space=pltpu.VMEM),
        ],
        out_specs=pl.BlockSpec(memory_space=pltpu.VMEM),
        scratch_shapes=[pltpu.VMEM((_B, _RP, _RP), jnp.float32)],
    )(roi, roi_t)

    sup_words = pl.kernel(
        _sc_scan_body,
        out_type=jax.ShapeDtypeStruct((_B, 32), jnp.int32),
        mesh=plsc.VectorSubcoreMesh(core_axis_name="c", subcore_axis_name="s"),
        scratch_types=[pltpu.VMEM((128, 32), jnp.int32),
                       pltpu.VMEM((32,), jnp.int32)],
    )(mp)

    roi_clipped = pl.pallas_call(
        _select_body,
        out_shape=jax.ShapeDtypeStruct((_B, _NKEEP, 4), jnp.int32),
        in_specs=[
            pl.BlockSpec(memory_space=pltpu.VMEM),
            pl.BlockSpec(memory_space=pltpu.VMEM),
        ],
        out_specs=pl.BlockSpec(memory_space=pltpu.VMEM),
    )(sup_words, roi_t)


    roi_flat = roi_clipped.reshape(-1)

    pooled = pl.pallas_call(
        _pool_body,
        grid=(_B * _NKEEP,),
        out_shape=jax.ShapeDtypeStruct((_B, _NKEEP, _POOL, _POOL, _C),
                                       jnp.float32),
        in_specs=[
            pl.BlockSpec(memory_space=pltpu.SMEM),
            pl.BlockSpec(memory_space=pl.ANY),
        ],
        out_specs=pl.BlockSpec(
            (1, 1, _POOL, _POOL, _C),
            lambda n: (n // _NKEEP, n % _NKEEP, 0, 0, 0)),
        scratch_shapes=[
            pltpu.VMEM((2, _TILE_R, _TILE_C, _C), jnp.float32),
            pltpu.SemaphoreType.DMA((2,)),
        ],
        compiler_params=pltpu.CompilerParams(
            dimension_semantics=("arbitrary",)),
    )(roi_flat, features)

    return pooled, roi_clipped


# speedup vs baseline: 1.2685x; 1.0251x over previous
import jax
import jax.numpy as jnp
from jax import lax
from jax.experimental import pallas as pl
from jax.experimental.pallas import tpu as pltpu
from jax.experimental.pallas import tpu_sc as plsc

_B, _R = 4, 1000
_RP = 1024
_NKEEP = 64
_H, _W, _C = 200, 200, 256
_TILE_R = 64
_TILE_C = 72
_POOL = 3


def _nms_build_body(roi_s_ref, roi_t_ref, mp_ref, m_ref):
    f32 = jnp.float32
    xj = roi_t_ref[:, 0:1, :]
    yj = roi_t_ref[:, 1:2, :]
    wj = roi_t_ref[:, 2:3, :]
    hj = roi_t_ref[:, 3:4, :]
    jlane = lax.broadcasted_iota(jnp.int32, (_B, 8, _RP), 2)

    def mk_chunk(ci, carry):
        c0 = ci * 8
        blk = roi_s_ref[:, pl.ds(c0, 8), :]
        xi = blk[:, :, 0:1]
        yi = blk[:, :, 1:2]
        wi = blk[:, :, 2:3]
        hi = blk[:, :, 3:4]
        x1 = jnp.maximum(xi, xj)
        y1 = jnp.maximum(yi, yj)
        x2 = jnp.minimum(xi + wi, xj + wj)
        y2 = jnp.minimum(yi + hi, yj + hj)
        inter = jnp.maximum(0.0, x2 - x1) * jnp.maximum(0.0, y2 - y1)
        union = wi * hi + wj * hj - inter
        iou = inter / union
        isub = c0 + lax.broadcasted_iota(jnp.int32, (_B, 8, _RP), 1)
        m = jnp.where((iou > 0.4) & (jlane > isub) & (jlane < _R),
                      f32(1.0), f32(0.0))
        m_ref[:, pl.ds(c0, 8), :] = m
        return carry

    lax.fori_loop(0, _R // 8, mk_chunk, 0)

    pj = lax.broadcasted_iota(jnp.int32, (_RP, 32), 0)
    pt = lax.broadcasted_iota(jnp.int32, (_RP, 32), 1)
    word = pj // 32
    bit = pj % 32
    sel_lo = (word == pt) & (bit < 16)
    sel_hi = (word == pt) & (bit >= 16)
    pmat_lo = jnp.where(sel_lo, 1 << jnp.where(sel_lo, bit, 0),
                        0).astype(f32)
    pmat_hi = jnp.where(sel_hi, 1 << jnp.where(sel_hi, bit - 16, 0),
                        0).astype(f32)

    def pack_slab(s, carry):
        for b in range(_B):
            slab = m_ref[b, pl.ds(s * 128, 128), :]
            lo = lax.dot_general(
                slab, pmat_lo, (((1,), (0,)), ((), ())),
                preferred_element_type=f32).astype(jnp.int32)
            hi = lax.dot_general(
                slab, pmat_hi, (((1,), (0,)), ((), ())),
                preferred_element_type=f32).astype(jnp.int32)
            mp_ref[pl.ds(b, 1), pl.ds(s * 128, 128), :] = (
                (lo | (hi << 16)).reshape(1, 128, 32))
        return carry

    lax.fori_loop(0, _RP // 128, pack_slab, 0)


def _dyn_gather16(v, idx):
    dnums = lax.GatherDimensionNumbers(
        offset_dims=(), collapsed_slice_dims=(0,), start_index_map=(0,))
    return lax.gather(v, idx[:, None], dnums, (1,),
                      mode=lax.GatherScatterMode.PROMISE_IN_BOUNDS)


def _sc_scan_body(mp_hbm, sup_hbm, mrows_v, supout_v):
    cid = lax.axis_index("c")
    sid = lax.axis_index("s")

    @pl.when((cid == 0) & (sid < _B))
    def _():
        unroll = 128

        def chunk(i, carry):
            s0, s1 = carry
            pltpu.sync_copy(mp_hbm.at[sid, pl.ds(i * unroll, unroll)],
                            mrows_v)
            for t in range(unroll):
                p = i * unroll + t
                row0 = mrows_v[t, pl.ds(0, 16)]
                row1 = mrows_v[t, pl.ds(16, 16)]
                wp = i * (unroll // 32) + t // 32
                wpv = jnp.full((16,), wp, jnp.int32)
                w0v = _dyn_gather16(s0, jnp.minimum(wpv, 15))
                w1v = _dyn_gather16(s1, jnp.maximum(wpv - 16, 0))
                a = ((wpv - 16) >> 31) & 1
                wv = w0v * a + w1v * (1 - a)
                bp = t % 32
                kb = 1 - ((wv >> bp) & 1)
                pb = jnp.full((16,), p, jnp.int32)
                inb = ((pb - (_R - 1)) >> 31) & 1
                m = kb * inb
                s0 = s0 | (row0 * m)
                s1 = s1 | (row1 * m)
            return s0, s1

        z = jnp.zeros((16,), jnp.int32)
        s0, s1 = lax.fori_loop(0, _RP // unroll, chunk, (z, z))
        supout_v[pl.ds(0, 16)] = s0
        supout_v[pl.ds(16, 16)] = s1
        pltpu.sync_copy(supout_v, sup_hbm.at[sid])


def _select_body(sup_ref, roi_t_ref, out_ref):
    f32 = jnp.float32
    sup = sup_ref[...]

    parts = [jnp.broadcast_to(sup[:, t:t + 1], (_B, 32)) for t in range(32)]
    rep = jnp.concatenate(parts, axis=1)
    jl2 = lax.broadcasted_iota(jnp.int32, (_B, _RP), 1)
    supbit = (rep >> (jl2 % 32)) & 1

    valid = jnp.where((jl2 < _R) & (supbit == 0), f32(1.0), f32(0.0))
    c = valid
    sh = 1
    while sh < _RP:
        z = jnp.zeros((_B, sh), f32)
        c = c + jnp.concatenate([z, c[:, : _RP - sh]], axis=1)
        sh *= 2
    rank0 = c - 1.0

    kio = lax.broadcasted_iota(jnp.int32, (_B, _NKEEP, _RP), 1)
    rank0i = rank0.astype(jnp.int32)
    oh = jnp.where(
        (valid[:, None, :] > 0.5) & (rank0i[:, None, :] == kio),
        f32(1.0), f32(0.0))
    found = jnp.sum(oh, axis=2)

    comps = []
    for d in range(4):
        cj = roi_t_ref[:, d:d + 1, :]
        s = jnp.sum(oh * cj, axis=2)
        s = jnp.where(found > 0.5, s, cj[:, :, _R - 1])
        comps.append(s)
    bx, by, bw, bh = comps

    xmin = jnp.maximum(0.0, bx).astype(jnp.int32)
    ymin = jnp.maximum(0.0, by).astype(jnp.int32)
    xmax = jnp.minimum(float(_W), bx + bw).astype(jnp.int32)
    ymax = jnp.minimum(float(_H), by + bh).astype(jnp.int32)

    def clip_axis(mn, mx, size, ps):
        pad = ps - (mx - mn)
        fix_min = mn < pad // 2
        fix_max = size - mx < (1 + pad) // 2
        pos = pad > 0
        sym = pos & (~(fix_min | fix_max))
        omin = jnp.where(sym, mn - pad // 2, mn)
        omax = jnp.where(sym, mx + (1 + pad) // 2, mx)
        omin = jnp.where(pos & fix_min, 0, omin)
        omax = jnp.where(pos & fix_min, ps, omax)
        omin = jnp.where(pos & fix_max, size - ps, omin)
        omax = jnp.where(pos & fix_max, size, omax)
        return omin, omax

    xo0, xo1 = clip_axis(xmin, xmax, _W, _POOL)
    yo0, yo1 = clip_axis(ymin, ymax, _H, _POOL)
    out_ref[...] = jnp.stack([xo0, yo0, xo1 - xo0, yo1 - yo0], axis=-1)


def _pool_body(roi_smem, feat_hbm, out_ref, buf, sem):
    n = pl.program_id(0)
    neg = jnp.float32(-jnp.inf)

    def coords(m):
        base = m * 4
        return (m // _NKEEP, roi_smem[base], roi_smem[base + 1],
                roi_smem[base + 2], roi_smem[base + 3])

    def dma_chunks(m, slot):
        b, x, y, w, h = coords(m)
        yc = jnp.minimum(y, _H - _TILE_R)
        xc = jnp.minimum((x // 8) * 8, _W - _TILE_C)
        dy = y - yc
        dx = x - xc
        out = []
        for k in range(_TILE_R // 8):
            for c in range(_TILE_C // 24):
                cond = ((8 * k < dy + h) & (8 * k + 8 > dy)
                        & (24 * c < dx + w) & (24 * c + 24 > dx))
                desc = pltpu.make_async_copy(
                    feat_hbm.at[b, pl.ds(yc + 8 * k, 8),
                                pl.ds(xc + 24 * c, 24), :],
                    buf.at[slot, pl.ds(8 * k, 8), pl.ds(24 * c, 24), :],
                    sem.at[slot])
                out.append((cond, desc))
        return out

    @pl.when(n == 0)
    def _():
        for cond, desc in dma_chunks(0, 0):
            @pl.when(cond)
            def _(desc=desc):
                desc.start()

    @pl.when(n + 1 < _B * _NKEEP)
    def _():
        for cond, desc in dma_chunks(n + 1, (n + 1) % 2):
            @pl.when(cond)
            def _(desc=desc):
                desc.start()

    for cond, desc in dma_chunks(n, n % 2):
        @pl.when(cond)
        def _(desc=desc):
            desc.wait()

    slot = n % 2
    b, x, y, w, h = coords(n)
    yc = jnp.minimum(y, _H - _TILE_R)
    xc = jnp.minimum((x // 8) * 8, _W - _TILE_C)
    dy = y - yc
    dx = x - xc
    hs = h // _POOL
    ws = w // _POOL
    rlo = (dy, dy + hs, dy + 2 * hs)
    rhi = (dy + hs, dy + 2 * hs, dy + h)
    clo = (dx, dx + ws, dx + 2 * ws)
    chi = (dx + ws, dx + 2 * ws, dx + w)

    acc0 = jnp.full((_TILE_C, _C), neg, jnp.float32)

    def row_max(r, a):
        return jnp.maximum(a, buf[slot, r])

    cio = lax.broadcasted_iota(jnp.int32, (_TILE_C, 1), 0)
    for i in range(3):
        rows_i = lax.fori_loop(rlo[i], rhi[i], row_max, acc0)
        for j in range(3):
            cm = (cio >= clo[j]) & (cio < chi[j])
            out_ref[0, 0, i, j] = jnp.max(jnp.where(cm, rows_i, neg), axis=0)


def kernel(features, roi):
    roi_t = jnp.transpose(roi, (0, 2, 1))
    roi_t = jnp.pad(roi_t, ((0, 0), (0, 0), (0, _RP - _R)))

    mp = pl.pallas_call(
        _nms_build_body,
        out_shape=jax.ShapeDtypeStruct((_B, _RP, 32), jnp.int32),
        in_specs=[
            pl.BlockSpec(memory_space=pltpu.VMEM),
            pl.BlockSpec(memory_space=pltpu.VMEM),
        ],
        out_specs=pl.BlockSpec(memory_space=pltpu.VMEM),
        scratch_shapes=[pltpu.VMEM((_B, _RP, _RP), jnp.float32)],
    )(roi, roi_t)

    sup_words = pl.kernel(
        _sc_scan_body,
        out_type=jax.ShapeDtypeStruct((_B, 32), jnp.int32),
        mesh=plsc.VectorSubcoreMesh(core_axis_name="c", subcore_axis_name="s"),
        scratch_types=[pltpu.VMEM((128, 32), jnp.int32),
                       pltpu.VMEM((32,), jnp.int32)],
    )(mp)

    roi_clipped = pl.pallas_call(
        _select_body,
        out_shape=jax.ShapeDtypeStruct((_B, _NKEEP, 4), jnp.int32),
        in_specs=[
            pl.BlockSpec(memory_space=pltpu.VMEM),
            pl.BlockSpec(memory_space=pltpu.VMEM),
        ],
        out_specs=pl.BlockSpec(memory_space=pltpu.VMEM),
    )(sup_words, roi_t)


    roi_flat = roi_clipped.reshape(-1)

    pooled = pl.pallas_call(
        _pool_body,
        grid=(_B * _NKEEP,),
        out_shape=jax.ShapeDtypeStruct((_B, _NKEEP, _POOL, _POOL, _C),
                                       jnp.float32),
        in_specs=[
            pl.BlockSpec(memory_space=pltpu.SMEM),
            pl.BlockSpec(memory_space=pl.ANY),
        ],
        out_specs=pl.BlockSpec(
            (1, 1, _POOL, _POOL, _C),
            lambda n: (n // _NKEEP, n % _NKEEP, 0, 0, 0)),
        scratch_shapes=[
            pltpu.VMEM((2, _TILE_R, _TILE_C, _C), jnp.float32),
            pltpu.SemaphoreType.DMA((2,)),
        ],
        compiler_params=pltpu.CompilerParams(
            dimension_semantics=("arbitrary",)),
    )(roi_flat, features)

    return pooled, roi_clipped
